# baseline (speedup 1.0000x reference)
import functools

import jax
import jax.numpy as jnp
from jax import lax
from jax.experimental import pallas as pl
from jax.experimental.pallas import tpu as pltpu
from jax.experimental.pallas import tpu_sc as plsc

N = 10000
E = 320000
NODE_DIM = 128
EDGE_DIM = 16
HIDDEN = 64
OUT_DIM = 21

NC = 2
NS = 16
NW = NC * NS
GROUP = 128
ROWS_PER_SUB = 80
R_PAD = NW * ROWS_PER_SUB
E_PAD = R_PAD * GROUP
N_PAD = 10112
ZROWS = N_PAD // NS
BE = 2560
BN = 2000


def _leaky(v):
    return jnp.where(v > 0, v, 0.2 * v)


def _node_encode_body(x_ref, w_ref, b_ref, o_ref):
    h = jnp.dot(x_ref[...], w_ref[...], preferred_element_type=jnp.float32)
    o_ref[...] = _leaky(h + b_ref[...])


def _edge_encode_body(ea_ref, we_ref, be_ref, w1_ref, b1_ref, w2_ref, b2_ref,
                      o1_ref, o2_ref):
    e = jnp.dot(ea_ref[...], we_ref[...], preferred_element_type=jnp.float32)
    e = _leaky(e + be_ref[...])
    o1_ref[...] = jnp.dot(e, w1_ref[...], preferred_element_type=jnp.float32) + b1_ref[...]
    o2_ref[...] = jnp.dot(e, w2_ref[...], preferred_element_type=jnp.float32) + b2_ref[...]


def _node_mlp_body(h_ref, a0_ref, a1_ref, w1_ref, b1_ref, w2_ref, b2_ref, o_ref):
    t = h_ref[...] + a0_ref[...] + a1_ref[...]
    t = _leaky(jnp.dot(t, w1_ref[...], preferred_element_type=jnp.float32) + b1_ref[...])
    t = jnp.dot(t, w2_ref[...], preferred_element_type=jnp.float32) + b2_ref[...]
    o_ref[...] = _leaky(t)


def _node_mlp_out_body(h_ref, a0_ref, a1_ref, w1_ref, b1_ref, w2_ref, b2_ref,
                       wo_ref, bo_ref, o_ref):
    t = h_ref[...] + a0_ref[...] + a1_ref[...]
    t = _leaky(jnp.dot(t, w1_ref[...], preferred_element_type=jnp.float32) + b1_ref[...])
    t = jnp.dot(t, w2_ref[...], preferred_element_type=jnp.float32) + b2_ref[...]
    t = _leaky(t)
    o_ref[...] = jnp.dot(t, wo_ref[...], preferred_element_type=jnp.float32) + bo_ref[...]


def _full(shape):
    return pl.BlockSpec(shape, lambda i: (0, 0))


def _rows(bs, width):
    return pl.BlockSpec((bs, width), lambda i: (i, 0))


def _node_encode(x, w, b):
    return pl.pallas_call(
        _node_encode_body,
        grid=(N // BN,),
        in_specs=[_rows(BN, NODE_DIM), _full((NODE_DIM, HIDDEN)), _full((1, HIDDEN))],
        out_specs=_rows(BN, HIDDEN),
        out_shape=jax.ShapeDtypeStruct((N, HIDDEN), jnp.float32),
    )(x, w, b)


def _edge_encode(ea, we, be, w1, b1, w2, b2):
    return pl.pallas_call(
        _edge_encode_body,
        grid=(E // BE,),
        in_specs=[_rows(BE, EDGE_DIM), _full((EDGE_DIM, HIDDEN)), _full((1, HIDDEN)),
                  _full((HIDDEN, HIDDEN)), _full((1, HIDDEN)),
                  _full((HIDDEN, HIDDEN)), _full((1, HIDDEN))],
        out_specs=[_rows(BE, HIDDEN), _rows(BE, HIDDEN)],
        out_shape=[jax.ShapeDtypeStruct((E_PAD, HIDDEN), jnp.float32),
                   jax.ShapeDtypeStruct((E_PAD, HIDDEN), jnp.float32)],
    )(ea, we, be, w1, b1, w2, b2)


def _node_mlp(h, a0, a1, w1, b1, w2, b2):
    return pl.pallas_call(
        _node_mlp_body,
        grid=(N // BN,),
        in_specs=[_rows(BN, HIDDEN)] * 3
        + [_full((HIDDEN, HIDDEN)), _full((1, HIDDEN)),
           _full((HIDDEN, HIDDEN)), _full((1, HIDDEN))],
        out_specs=_rows(BN, HIDDEN),
        out_shape=jax.ShapeDtypeStruct((N, HIDDEN), jnp.float32),
    )(h, a0, a1, w1, b1, w2, b2)


def _node_mlp_out(h, a0, a1, w1, b1, w2, b2, wo, bo):
    return pl.pallas_call(
        _node_mlp_out_body,
        grid=(N // BN,),
        in_specs=[_rows(BN, HIDDEN)] * 3
        + [_full((HIDDEN, HIDDEN)), _full((1, HIDDEN)),
           _full((HIDDEN, HIDDEN)), _full((1, HIDDEN)),
           _full((HIDDEN, 128)), _full((1, 128))],
        out_specs=_rows(BN, 128),
        out_shape=jax.ShapeDtypeStruct((N, 128), jnp.float32),
    )(h, a0, a1, w1, b1, w2, b2, wo, bo)


NB = 4
NSTEP = ROWS_PER_SUB // NB


def _sc_body(h_hbm, el_hbm, src_hbm, dst_hbm, out_hbm, *scr):
    isrc = scr[0]
    idst = scr[1]
    el = scr[2:6]
    g = scr[6:10]
    sem_s = scr[10]
    sem_d = scr[11]
    es = scr[12:16]
    gs = scr[16:20]
    ss = scr[20:24]
    agg = scr[24]

    c = lax.axis_index("c")
    s = lax.axis_index("s")

    base = (c * NS + s) * ROWS_PER_SUB

    pltpu.async_copy(src_hbm.at[pl.ds(base, ROWS_PER_SUB)], isrc, sem_s)
    pltpu.async_copy(dst_hbm.at[pl.ds(base, ROWS_PER_SUB)], idst, sem_d)

    zt = g[3]

    def zbody(i, carry):
        for k in range(HIDDEN // 16):
            zt[i, pl.ds(k * 16, 16)] = jnp.zeros((16,), jnp.float32)
        return carry
    lax.fori_loop(0, GROUP, zbody, 0, unroll=2)
    for k in range(ZROWS // GROUP):
        pltpu.sync_copy(zt, agg.at[pl.ds(s * ZROWS + k * GROUP, GROUP)])
    _zrem = ZROWS % GROUP
    if _zrem:
        pltpu.sync_copy(zt.at[pl.ds(0, _zrem)],
                        agg.at[pl.ds(s * ZROWS + (ZROWS // GROUP) * GROUP, _zrem)])

    pltpu.make_async_copy(src_hbm.at[pl.ds(0, ROWS_PER_SUB)], isrc, sem_s).wait()
    pltpu.make_async_copy(dst_hbm.at[pl.ds(0, ROWS_PER_SUB)], idst, sem_d).wait()
    plsc.subcore_barrier()

    def start(r, bi):
        pltpu.async_copy(el_hbm.at[pl.ds((base + r) * GROUP, GROUP)], el[bi], es[bi])
        pltpu.async_copy(h_hbm.at[isrc.at[r]], g[bi], gs[bi])

    def wait_inputs(b):
        pltpu.make_async_copy(el_hbm.at[pl.ds(0, GROUP)], el[b], es[b]).wait()
        pltpu.make_async_copy(h_hbm.at[isrc.at[0]], g[b], gs[b]).wait()

    def drain_scatter(bb):
        pltpu.make_async_copy(g[bb], agg.at[idst.at[0]], ss[bb]).wait()

    def compute(b):
        def cbody(i, carry):
            for k in range(HIDDEN // 16):
                sl = pl.ds(k * 16, 16)
                g[b][i, sl] = jnp.maximum(g[b][i, sl] + el[b][i, sl], 0.0)
            return carry
        lax.fori_loop(0, GROUP, cbody, 0, unroll=2)

    start(0, 0)
    start(1, 1)

    def step_body(t, carry):
        for b in range(NB):
            r = t * NB + b
            nb = (b + 2) % NB
            wait_inputs(b)
            compute(b)
            pltpu.async_copy(g[b], agg.at[idst.at[r]], ss[b], add=True)
            if b >= 2:
                drain_scatter(nb)
                @pl.when(t < NSTEP - 1)
                def _():
                    start(r + 2, nb)
            else:
                @pl.when(t > 0)
                def _():
                    drain_scatter(nb)
                start(r + 2, nb)
        return carry

    lax.fori_loop(0, NSTEP, step_body, 0, unroll=False)
    drain_scatter(2)
    drain_scatter(3)

    plsc.subcore_barrier()
    pltpu.sync_copy(agg.at[pl.ds(s * ZROWS, ZROWS)],
                    out_hbm.at[pl.ds((c * N_PAD) + s * ZROWS, ZROWS)])


def _sc_aggregate(h, el, src2d, dst2d):
    return pl.kernel(
        _sc_body,
        out_type=jax.ShapeDtypeStruct((NC * N_PAD, HIDDEN), jnp.float32),
        mesh=plsc.VectorSubcoreMesh(core_axis_name="c", subcore_axis_name="s"),
        compiler_params=pltpu.CompilerParams(use_tc_tiling_on_sc=False),
        scratch_types=(
            [pltpu.VMEM((ROWS_PER_SUB, GROUP), jnp.int32)] * 2
            + [pltpu.VMEM((GROUP, HIDDEN), jnp.float32)] * 4
            + [pltpu.VMEM((GROUP, HIDDEN), jnp.float32)] * 4
            + [pltpu.SemaphoreType.DMA] * 14
            + [pltpu.VMEM_SHARED((N_PAD, HIDDEN), jnp.float32)]
        ),
    )(h, el, src2d, dst2d)


def kernel(x, edge_attr, edge_index, W_node, b_node, W_edge, b_edge,
           c1_lw, c1_lb, c1_w1, c1_b1, c1_w2, c1_b2,
           c2_lw, c2_lb, c2_w1, c2_b1, c2_w2, c2_b2,
           W_out, b_out):
    f32 = jnp.float32
    pad_e = E_PAD - E
    src2d = jnp.concatenate([edge_index[0], jnp.zeros((pad_e,), jnp.int32)]
                            ).reshape(R_PAD, GROUP)
    dst2d = jnp.concatenate([edge_index[1], jnp.full((pad_e,), N, jnp.int32)]
                            ).reshape(R_PAD, GROUP)

    b_node2 = b_node.reshape(1, HIDDEN)
    b_edge2 = b_edge.reshape(1, HIDDEN)
    wo_p = jnp.zeros((HIDDEN, 128), f32).at[:, :OUT_DIM].set(W_out)
    bo_p = jnp.zeros((1, 128), f32).at[0, :OUT_DIM].set(b_out)

    h0 = _node_encode(x, W_node, b_node2)
    el1, el2 = _edge_encode(edge_attr, W_edge, b_edge2, c1_lw, c1_lb.reshape(1, HIDDEN),
                            c2_lw, c2_lb.reshape(1, HIDDEN))

    agg = _sc_aggregate(h0, el1, src2d, dst2d)
    h1 = _node_mlp(h0, agg[:N], agg[N_PAD:N_PAD + N],
                   c1_w1, c1_b1.reshape(1, HIDDEN), c1_w2, c1_b2.reshape(1, HIDDEN))

    agg2 = _sc_aggregate(h1, el2, src2d, dst2d)
    out_p = _node_mlp_out(h1, agg2[:N], agg2[N_PAD:N_PAD + N],
                          c2_w1, c2_b1.reshape(1, HIDDEN), c2_w2, c2_b2.reshape(1, HIDDEN),
                          wo_p, bo_p)
    return out_p[:, :OUT_DIM]


# speedup vs baseline: 1.3062x; 1.3062x over previous
import functools

import jax
import jax.numpy as jnp
from jax import lax
from jax.experimental import pallas as pl
from jax.experimental.pallas import tpu as pltpu
from jax.experimental.pallas import tpu_sc as plsc

N = 10000
E = 320000
NODE_DIM = 128
EDGE_DIM = 16
HIDDEN = 64
OUT_DIM = 21

NC = 2
NS = 16
NW = NC * NS
GROUP = 128
ROWS_PER_SUB = 80
R_PAD = NW * ROWS_PER_SUB
E_PAD = R_PAD * GROUP
N_PAD = 10112
ZROWS = N_PAD // NS
BE = 2560
BN = 2000


def _leaky(v):
    return jnp.where(v > 0, v, 0.2 * v)


def _node_encode_body(x_ref, w_ref, b_ref, o_ref):
    h = jnp.dot(x_ref[...], w_ref[...], preferred_element_type=jnp.float32)
    o_ref[...] = _leaky(h + b_ref[...])


def _edge_encode_body(ea_ref, we_ref, be_ref, w1_ref, b1_ref, w2_ref, b2_ref,
                      o1_ref, o2_ref):
    e = jnp.dot(ea_ref[...], we_ref[...], preferred_element_type=jnp.float32)
    e = _leaky(e + be_ref[...])
    o1_ref[...] = jnp.dot(e, w1_ref[...], preferred_element_type=jnp.float32) + b1_ref[...]
    o2_ref[...] = jnp.dot(e, w2_ref[...], preferred_element_type=jnp.float32) + b2_ref[...]


def _node_mlp_body(h_ref, a0_ref, a1_ref, w1_ref, b1_ref, w2_ref, b2_ref, o_ref):
    t = h_ref[...] + a0_ref[...] + a1_ref[...]
    t = _leaky(jnp.dot(t, w1_ref[...], preferred_element_type=jnp.float32) + b1_ref[...])
    t = jnp.dot(t, w2_ref[...], preferred_element_type=jnp.float32) + b2_ref[...]
    o_ref[...] = _leaky(t)


def _node_mlp_out_body(h_ref, a0_ref, a1_ref, w1_ref, b1_ref, w2_ref, b2_ref,
                       wo_ref, bo_ref, o_ref):
    t = h_ref[...] + a0_ref[...] + a1_ref[...]
    t = _leaky(jnp.dot(t, w1_ref[...], preferred_element_type=jnp.float32) + b1_ref[...])
    t = jnp.dot(t, w2_ref[...], preferred_element_type=jnp.float32) + b2_ref[...]
    t = _leaky(t)
    o_ref[...] = jnp.dot(t, wo_ref[...], preferred_element_type=jnp.float32) + bo_ref[...]


def _full(shape):
    return pl.BlockSpec(shape, lambda i: (0, 0))


def _rows(bs, width):
    return pl.BlockSpec((bs, width), lambda i: (i, 0))


def _node_encode(x, w, b):
    return pl.pallas_call(
        _node_encode_body,
        grid=(N // BN,),
        in_specs=[_rows(BN, NODE_DIM), _full((NODE_DIM, HIDDEN)), _full((1, HIDDEN))],
        out_specs=_rows(BN, HIDDEN),
        out_shape=jax.ShapeDtypeStruct((N, HIDDEN), jnp.float32),
    )(x, w, b)


def _edge_encode(ea, we, be, w1, b1, w2, b2):
    return pl.pallas_call(
        _edge_encode_body,
        grid=(E // BE,),
        in_specs=[_rows(BE, EDGE_DIM), _full((EDGE_DIM, HIDDEN)), _full((1, HIDDEN)),
                  _full((HIDDEN, HIDDEN)), _full((1, HIDDEN)),
                  _full((HIDDEN, HIDDEN)), _full((1, HIDDEN))],
        out_specs=[_rows(BE, HIDDEN), _rows(BE, HIDDEN)],
        out_shape=[jax.ShapeDtypeStruct((E, HIDDEN), jnp.float32),
                   jax.ShapeDtypeStruct((E, HIDDEN), jnp.float32)],
    )(ea, we, be, w1, b1, w2, b2)


def _node_mlp(h, a0, a1, w1, b1, w2, b2):
    return pl.pallas_call(
        _node_mlp_body,
        grid=(N // BN,),
        in_specs=[_rows(BN, HIDDEN)] * 3
        + [_full((HIDDEN, HIDDEN)), _full((1, HIDDEN)),
           _full((HIDDEN, HIDDEN)), _full((1, HIDDEN))],
        out_specs=_rows(BN, HIDDEN),
        out_shape=jax.ShapeDtypeStruct((N, HIDDEN), jnp.float32),
    )(h, a0, a1, w1, b1, w2, b2)


def _node_mlp_out(h, a0, a1, w1, b1, w2, b2, wo, bo):
    return pl.pallas_call(
        _node_mlp_out_body,
        grid=(N // BN,),
        in_specs=[_rows(BN, HIDDEN)] * 3
        + [_full((HIDDEN, HIDDEN)), _full((1, HIDDEN)),
           _full((HIDDEN, HIDDEN)), _full((1, HIDDEN)),
           _full((HIDDEN, 128)), _full((1, 128))],
        out_specs=_rows(BN, 128),
        out_shape=jax.ShapeDtypeStruct((N, 128), jnp.float32),
    )(h, a0, a1, w1, b1, w2, b2, wo, bo)


NB = 4
NSTEP = ROWS_PER_SUB // NB


EDGES_PER_SUB = ROWS_PER_SUB * GROUP
MAX_OFF_G = (E - EDGES_PER_SUB) // GROUP


def _sc_body(h_hbm, el_hbm, ei_hbm, out_hbm, *scr):
    isrc = scr[0]
    idst = scr[1]
    el = scr[2:6]
    g = scr[6:10]
    sem_s = scr[10]
    sem_d = scr[11]
    es = scr[12:16]
    gs = scr[16:20]
    ss = scr[20:24]
    agg = scr[24]

    c = lax.axis_index("c")
    s = lax.axis_index("s")

    base = (c * NS + s) * ROWS_PER_SUB
    off_g = jnp.minimum(base, MAX_OFF_G)
    r0 = base - off_g
    nsteps = (ROWS_PER_SUB - r0) // NB

    pltpu.async_copy(ei_hbm.at[0].at[pl.ds(off_g * GROUP, EDGES_PER_SUB)], isrc, sem_s)
    pltpu.async_copy(ei_hbm.at[1].at[pl.ds(off_g * GROUP, EDGES_PER_SUB)], idst, sem_d)

    zt = g[3]

    def zbody(i, carry):
        for k in range(HIDDEN // 16):
            zt[i, pl.ds(k * 16, 16)] = jnp.zeros((16,), jnp.float32)
        return carry
    lax.fori_loop(0, GROUP, zbody, 0, unroll=2)
    for k in range(ZROWS // GROUP):
        pltpu.sync_copy(zt, agg.at[pl.ds(s * ZROWS + k * GROUP, GROUP)])
    _zrem = ZROWS % GROUP
    if _zrem:
        pltpu.sync_copy(zt.at[pl.ds(0, _zrem)],
                        agg.at[pl.ds(s * ZROWS + (ZROWS // GROUP) * GROUP, _zrem)])

    pltpu.make_async_copy(ei_hbm.at[0].at[pl.ds(0, EDGES_PER_SUB)], isrc, sem_s).wait()
    pltpu.make_async_copy(ei_hbm.at[1].at[pl.ds(0, EDGES_PER_SUB)], idst, sem_d).wait()
    plsc.subcore_barrier()

    def start(r, bi):
        pltpu.async_copy(el_hbm.at[pl.ds((off_g + r) * GROUP, GROUP)], el[bi], es[bi])
        pltpu.async_copy(h_hbm.at[isrc.at[pl.ds(r * GROUP, GROUP)]], g[bi], gs[bi])

    def wait_inputs(b):
        pltpu.make_async_copy(el_hbm.at[pl.ds(0, GROUP)], el[b], es[b]).wait()
        pltpu.make_async_copy(h_hbm.at[isrc.at[pl.ds(0, GROUP)]], g[b], gs[b]).wait()

    def drain_scatter(bb):
        pltpu.make_async_copy(g[bb], agg.at[idst.at[pl.ds(0, GROUP)]], ss[bb]).wait()

    def compute(b):
        def cbody(i, carry):
            for k in range(HIDDEN // 16):
                sl = pl.ds(k * 16, 16)
                g[b][i, sl] = jnp.maximum(g[b][i, sl] + el[b][i, sl], 0.0)
            return carry
        lax.fori_loop(0, GROUP, cbody, 0, unroll=2)

    start(r0, 0)
    start(r0 + 1, 1)

    def step_body(t, carry):
        for b in range(NB):
            r = r0 + t * NB + b
            nb = (b + 2) % NB
            wait_inputs(b)
            compute(b)
            pltpu.async_copy(g[b], agg.at[idst.at[pl.ds(r * GROUP, GROUP)]], ss[b],
                             add=True)
            if b >= 2:
                drain_scatter(nb)
                @pl.when(t < nsteps - 1)
                def _():
                    start(r + 2, nb)
            else:
                @pl.when(t > 0)
                def _():
                    drain_scatter(nb)
                start(r + 2, nb)
        return carry

    lax.fori_loop(0, nsteps, step_body, 0, unroll=False)
    drain_scatter(2)
    drain_scatter(3)

    plsc.subcore_barrier()
    pltpu.sync_copy(agg.at[pl.ds(s * ZROWS, ZROWS)],
                    out_hbm.at[pl.ds((c * N_PAD) + s * ZROWS, ZROWS)])


def _sc_aggregate(h, el, edge_index):
    return pl.kernel(
        _sc_body,
        out_type=jax.ShapeDtypeStruct((NC * N_PAD, HIDDEN), jnp.float32),
        mesh=plsc.VectorSubcoreMesh(core_axis_name="c", subcore_axis_name="s"),
        compiler_params=pltpu.CompilerParams(use_tc_tiling_on_sc=False),
        scratch_types=(
            [pltpu.VMEM((EDGES_PER_SUB,), jnp.int32)] * 2
            + [pltpu.VMEM((GROUP, HIDDEN), jnp.float32)] * 4
            + [pltpu.VMEM((GROUP, HIDDEN), jnp.float32)] * 4
            + [pltpu.SemaphoreType.DMA] * 14
            + [pltpu.VMEM_SHARED((N_PAD, HIDDEN), jnp.float32)]
        ),
    )(h, el, edge_index)


def kernel(x, edge_attr, edge_index, W_node, b_node, W_edge, b_edge,
           c1_lw, c1_lb, c1_w1, c1_b1, c1_w2, c1_b2,
           c2_lw, c2_lb, c2_w1, c2_b1, c2_w2, c2_b2,
           W_out, b_out):
    f32 = jnp.float32
    b_node2 = b_node.reshape(1, HIDDEN)
    b_edge2 = b_edge.reshape(1, HIDDEN)
    wo_p = jnp.zeros((HIDDEN, 128), f32).at[:, :OUT_DIM].set(W_out)
    bo_p = jnp.zeros((1, 128), f32).at[0, :OUT_DIM].set(b_out)

    h0 = _node_encode(x, W_node, b_node2)
    el1, el2 = _edge_encode(edge_attr, W_edge, b_edge2, c1_lw, c1_lb.reshape(1, HIDDEN),
                            c2_lw, c2_lb.reshape(1, HIDDEN))

    agg = _sc_aggregate(h0, el1, edge_index)
    h1 = _node_mlp(h0, agg[:N], agg[N_PAD:N_PAD + N],
                   c1_w1, c1_b1.reshape(1, HIDDEN), c1_w2, c1_b2.reshape(1, HIDDEN))

    agg2 = _sc_aggregate(h1, el2, edge_index)
    out_p = _node_mlp_out(h1, agg2[:N], agg2[N_PAD:N_PAD + N],
                          c2_w1, c2_b1.reshape(1, HIDDEN), c2_w2, c2_b2.reshape(1, HIDDEN),
                          wo_p, bo_p)
    return out_p[:, :OUT_DIM]


# speedup vs baseline: 1.3961x; 1.0688x over previous
import functools

import jax
import jax.numpy as jnp
import numpy as np
from jax import lax
from jax.experimental import pallas as pl
from jax.experimental.pallas import tpu as pltpu
from jax.experimental.pallas import tpu_sc as plsc

N = 10000
E = 320000
NODE_DIM = 128
EDGE_DIM = 16
HIDDEN = 64
OUT_DIM = 21

NC = 2
NS = 16
NW = NC * NS
GROUP = 128
ROWS_PER_SUB = 80
R_PAD = NW * ROWS_PER_SUB
E_PAD = R_PAD * GROUP
N_PAD = 10112
ZROWS = N_PAD // NS
BE = 2560
BN = 2000


def _leaky(v):
    return jnp.where(v > 0, v, 0.2 * v)


def _node_encode_body(x_ref, w_ref, b_ref, o_ref):
    h = jnp.dot(x_ref[...], w_ref[...], preferred_element_type=jnp.float32)
    o_ref[...] = _leaky(h + b_ref[...])


def _edge_encode_body(ea_ref, we_ref, be_ref, w1_ref, b1_ref, w2_ref, b2_ref,
                      o_ref):
    e = jnp.dot(ea_ref[...], we_ref[...], preferred_element_type=jnp.float32)
    e = _leaky(e + be_ref[...])
    el1 = jnp.dot(e, w1_ref[...], preferred_element_type=jnp.float32) + b1_ref[...]
    el2 = jnp.dot(e, w2_ref[...], preferred_element_type=jnp.float32) + b2_ref[...]
    o_ref[...] = jnp.concatenate([el1, el2], axis=1).astype(jnp.bfloat16)


def _node_mlp_body(h_ref, a0_ref, a1_ref, w1_ref, b1_ref, w2_ref, b2_ref, o_ref):
    t = h_ref[...] + a0_ref[...] + a1_ref[...]
    t = _leaky(jnp.dot(t, w1_ref[...], preferred_element_type=jnp.float32) + b1_ref[...])
    t = jnp.dot(t, w2_ref[...], preferred_element_type=jnp.float32) + b2_ref[...]
    o_ref[...] = _leaky(t)


def _node_mlp_out_body(h_ref, a0_ref, a1_ref, w1_ref, b1_ref, w2_ref, b2_ref,
                       wo_ref, bo_ref, o_ref):
    t = h_ref[...] + a0_ref[...] + a1_ref[...]
    t = _leaky(jnp.dot(t, w1_ref[...], preferred_element_type=jnp.float32) + b1_ref[...])
    t = jnp.dot(t, w2_ref[...], preferred_element_type=jnp.float32) + b2_ref[...]
    t = _leaky(t)
    o_ref[...] = jnp.dot(t, wo_ref[...], preferred_element_type=jnp.float32) + bo_ref[...]


def _full(shape):
    return pl.BlockSpec(shape, lambda i: (0, 0))


def _rows(bs, width):
    return pl.BlockSpec((bs, width), lambda i: (i, 0))


def _node_encode(x, w, b):
    return pl.pallas_call(
        _node_encode_body,
        grid=(N // BN,),
        in_specs=[_rows(BN, NODE_DIM), _full((NODE_DIM, HIDDEN)), _full((1, HIDDEN))],
        out_specs=_rows(BN, HIDDEN),
        out_shape=jax.ShapeDtypeStruct((N, HIDDEN), jnp.float32),
    )(x, w, b)


def _edge_encode(ea, we, be, w1, b1, w2, b2):
    return pl.pallas_call(
        _edge_encode_body,
        grid=(E // BE,),
        in_specs=[_rows(BE, EDGE_DIM), _full((EDGE_DIM, HIDDEN)), _full((1, HIDDEN)),
                  _full((HIDDEN, HIDDEN)), _full((1, HIDDEN)),
                  _full((HIDDEN, HIDDEN)), _full((1, HIDDEN))],
        out_specs=_rows(BE, 2 * HIDDEN),
        out_shape=jax.ShapeDtypeStruct((E, 2 * HIDDEN), jnp.bfloat16),
    )(ea, we, be, w1, b1, w2, b2)


def _node_mlp(h, a0, a1, w1, b1, w2, b2):
    return pl.pallas_call(
        _node_mlp_body,
        grid=(N // BN,),
        in_specs=[_rows(BN, HIDDEN)] * 3
        + [_full((HIDDEN, HIDDEN)), _full((1, HIDDEN)),
           _full((HIDDEN, HIDDEN)), _full((1, HIDDEN))],
        out_specs=_rows(BN, HIDDEN),
        out_shape=jax.ShapeDtypeStruct((N, HIDDEN), jnp.float32),
    )(h, a0, a1, w1, b1, w2, b2)


def _node_mlp_out(h, a0, a1, w1, b1, w2, b2, wo, bo):
    return pl.pallas_call(
        _node_mlp_out_body,
        grid=(N // BN,),
        in_specs=[_rows(BN, HIDDEN)] * 3
        + [_full((HIDDEN, HIDDEN)), _full((1, HIDDEN)),
           _full((HIDDEN, HIDDEN)), _full((1, HIDDEN)),
           _full((HIDDEN, 128)), _full((1, 128))],
        out_specs=_rows(BN, 128),
        out_shape=jax.ShapeDtypeStruct((N, 128), jnp.float32),
    )(h, a0, a1, w1, b1, w2, b2, wo, bo)


NB = 4
NSTEP = ROWS_PER_SUB // NB


EDGES_PER_SUB = ROWS_PER_SUB * GROUP
MAX_OFF_G = (E - EDGES_PER_SUB) // GROUP

_PERM = np.empty((HIDDEN,), np.int32)
for _j in range(HIDDEN // 32):
    for _i in range(16):
        _PERM[32 * _j + 2 * _i] = 32 * _j + _i
        _PERM[32 * _j + 2 * _i + 1] = 32 * _j + 16 + _i


def _sc_body(h_hbm, el_hbm, ei_hbm, out_hbm, *scr, half):
    isrc = scr[0]
    idst = scr[1]
    el = scr[2:6]
    g = scr[6:10]
    sem_s = scr[10]
    sem_d = scr[11]
    es = scr[12:16]
    gs = scr[16:20]
    ss = scr[20:24]
    agg = scr[24]

    c = lax.axis_index("c")
    s = lax.axis_index("s")

    base = (c * NS + s) * ROWS_PER_SUB
    off_g = jnp.minimum(base, MAX_OFF_G)
    r0 = base - off_g
    nsteps = (ROWS_PER_SUB - r0) // NB

    pltpu.async_copy(ei_hbm.at[0].at[pl.ds(off_g * GROUP, EDGES_PER_SUB)], isrc, sem_s)
    pltpu.async_copy(ei_hbm.at[1].at[pl.ds(off_g * GROUP, EDGES_PER_SUB)], idst, sem_d)

    zt = g[3]

    def zbody(i, carry):
        for k in range(HIDDEN // 16):
            zt[i, pl.ds(k * 16, 16)] = jnp.zeros((16,), jnp.float32)
        return carry
    lax.fori_loop(0, GROUP, zbody, 0, unroll=2)
    for k in range(ZROWS // GROUP):
        pltpu.sync_copy(zt, agg.at[pl.ds(s * ZROWS + k * GROUP, GROUP)])
    _zrem = ZROWS % GROUP
    if _zrem:
        pltpu.sync_copy(zt.at[pl.ds(0, _zrem)],
                        agg.at[pl.ds(s * ZROWS + (ZROWS // GROUP) * GROUP, _zrem)])

    pltpu.make_async_copy(ei_hbm.at[0].at[pl.ds(0, EDGES_PER_SUB)], isrc, sem_s).wait()
    pltpu.make_async_copy(ei_hbm.at[1].at[pl.ds(0, EDGES_PER_SUB)], idst, sem_d).wait()
    plsc.subcore_barrier()

    hsl = pl.ds(half * HIDDEN, HIDDEN)

    def start(r, bi):
        pltpu.async_copy(el_hbm.at[pl.ds((off_g + r) * GROUP, GROUP), hsl],
                         el[bi], es[bi])
        pltpu.async_copy(h_hbm.at[isrc.at[pl.ds(r * GROUP, GROUP)]], g[bi], gs[bi])

    def wait_inputs(b):
        pltpu.make_async_copy(el_hbm.at[pl.ds(0, GROUP), hsl], el[b], es[b]).wait()
        pltpu.make_async_copy(h_hbm.at[isrc.at[pl.ds(0, GROUP)]], g[b], gs[b]).wait()

    def drain_scatter(bb):
        pltpu.make_async_copy(g[bb], agg.at[idst.at[pl.ds(0, GROUP)]], ss[bb]).wait()

    def compute(b):
        def cbody(i, carry):
            for j in range(HIDDEN // 32):
                v = el[b][i, pl.ds(32 * j, 32)]
                lo, hi = plsc.unpack(v, format=plsc.PackFormat.INTERLEAVED,
                                     preferred_element_type=jnp.float32)
                sl0 = pl.ds(32 * j, 16)
                sl1 = pl.ds(32 * j + 16, 16)
                g[b][i, sl0] = jnp.maximum(g[b][i, sl0] + lo, 0.0)
                g[b][i, sl1] = jnp.maximum(g[b][i, sl1] + hi, 0.0)
            return carry
        lax.fori_loop(0, GROUP, cbody, 0, unroll=2)

    start(r0, 0)
    start(r0 + 1, 1)

    def step_body(t, carry):
        for b in range(NB):
            r = r0 + t * NB + b
            nb = (b + 2) % NB
            wait_inputs(b)
            compute(b)
            pltpu.async_copy(g[b], agg.at[idst.at[pl.ds(r * GROUP, GROUP)]], ss[b],
                             add=True)
            if b >= 2:
                drain_scatter(nb)
                @pl.when(t < nsteps - 1)
                def _():
                    start(r + 2, nb)
            else:
                @pl.when(t > 0)
                def _():
                    drain_scatter(nb)
                start(r + 2, nb)
        return carry

    lax.fori_loop(0, nsteps, step_body, 0, unroll=False)
    drain_scatter(2)
    drain_scatter(3)

    plsc.subcore_barrier()
    pltpu.sync_copy(agg.at[pl.ds(s * ZROWS, ZROWS)],
                    out_hbm.at[pl.ds((c * N_PAD) + s * ZROWS, ZROWS)])


def _sc_aggregate(h, el, edge_index, half):
    return pl.kernel(
        functools.partial(_sc_body, half=half),
        out_type=jax.ShapeDtypeStruct((NC * N_PAD, HIDDEN), jnp.float32),
        mesh=plsc.VectorSubcoreMesh(core_axis_name="c", subcore_axis_name="s"),
        compiler_params=pltpu.CompilerParams(use_tc_tiling_on_sc=False,
                                             needs_layout_passes=False),
        scratch_types=(
            [pltpu.VMEM((EDGES_PER_SUB,), jnp.int32)] * 2
            + [pltpu.VMEM((GROUP, HIDDEN), jnp.bfloat16)] * 4
            + [pltpu.VMEM((GROUP, HIDDEN), jnp.float32)] * 4
            + [pltpu.SemaphoreType.DMA] * 14
            + [pltpu.VMEM_SHARED((N_PAD, HIDDEN), jnp.float32)]
        ),
    )(h, el, edge_index)


def kernel(x, edge_attr, edge_index, W_node, b_node, W_edge, b_edge,
           c1_lw, c1_lb, c1_w1, c1_b1, c1_w2, c1_b2,
           c2_lw, c2_lb, c2_w1, c2_b1, c2_w2, c2_b2,
           W_out, b_out):
    f32 = jnp.float32
    b_node2 = b_node.reshape(1, HIDDEN)
    b_edge2 = b_edge.reshape(1, HIDDEN)
    wo_p = jnp.zeros((HIDDEN, 128), f32).at[:, :OUT_DIM].set(W_out)
    bo_p = jnp.zeros((1, 128), f32).at[0, :OUT_DIM].set(b_out)

    h0 = _node_encode(x, W_node, b_node2)
    el = _edge_encode(edge_attr, W_edge, b_edge2,
                      c1_lw[:, _PERM], c1_lb[_PERM].reshape(1, HIDDEN),
                      c2_lw[:, _PERM], c2_lb[_PERM].reshape(1, HIDDEN))

    agg = _sc_aggregate(h0, el, edge_index, 0)
    h1 = _node_mlp(h0, agg[:N], agg[N_PAD:N_PAD + N],
                   c1_w1, c1_b1.reshape(1, HIDDEN), c1_w2, c1_b2.reshape(1, HIDDEN))

    agg2 = _sc_aggregate(h1, el, edge_index, 1)
    out_p = _node_mlp_out(h1, agg2[:N], agg2[N_PAD:N_PAD + N],
                          c2_w1, c2_b1.reshape(1, HIDDEN), c2_w2, c2_b2.reshape(1, HIDDEN),
                          wo_p, bo_p)
    return out_p[:, :OUT_DIM]
